# baseline (speedup 1.0000x reference)
import functools

import jax
import jax.numpy as jnp
from jax import lax
from jax.experimental import pallas as pl
from jax.experimental.pallas import tpu as pltpu
from jax.experimental.pallas import tpu_sc as plsc

NUM_EMB = 1000000
DIM = 32
ROW_BLK = 8000

_info = plsc.get_sparse_core_info()
_NC, _NS = _info.num_cores, _info.num_subcores
_NW = _NC * _NS

B_CH = 16


def _transform_body(xt_ref, w_ref, b_ref, y_ref):
    h = jax.lax.dot_general(
        xt_ref[...], w_ref[...], (((0,), (0,)), ((), ())),
        preferred_element_type=jnp.float32)
    h = h + b_ref[...]
    norm = jnp.sqrt(jnp.sum(h * h, axis=-1, keepdims=True))
    y = h / norm
    y_ref[...] = jnp.concatenate(
        [y, jnp.zeros((y.shape[0], 128 - DIM), jnp.float32)], axis=1)


COL_BLK = 16384


def _transform_table_t(table_t, W, b):
    return pl.pallas_call(
        _transform_body,
        grid=(pl.cdiv(NUM_EMB, COL_BLK),),
        in_specs=[
            pl.BlockSpec((DIM, COL_BLK), lambda i: (0, i)),
            pl.BlockSpec((DIM, DIM), lambda i: (0, 0)),
            pl.BlockSpec((1, DIM), lambda i: (0, 0)),
        ],
        out_specs=pl.BlockSpec((COL_BLK, 128), lambda i: (i, 0)),
        out_shape=jax.ShapeDtypeStruct((NUM_EMB, 128), jnp.float32),
    )(table_t, W, b.reshape(1, DIM))


def _make_gather3d(B, L):
    per_w = B // _NW
    n_ch = per_w // B_CH
    mesh = plsc.VectorSubcoreMesh(core_axis_name="c", subcore_axis_name="s")

    @functools.partial(
        pl.kernel,
        mesh=mesh,
        out_type=jax.ShapeDtypeStruct((L, DIM, B), jnp.float32),
        scratch_types=[
            pltpu.VMEM((B_CH, L), jnp.int32),
            pltpu.VMEM((B_CH, L), jnp.int32),
            pltpu.VMEM((B_CH * L, DIM), jnp.float32),
            pltpu.VMEM((B_CH * L, DIM), jnp.float32),
            pltpu.VMEM((L, DIM, B_CH), jnp.float32),
            pltpu.VMEM((L, DIM, B_CH), jnp.float32),
            pltpu.SemaphoreType.DMA,
            pltpu.SemaphoreType.DMA,
        ],
        compiler_params=pltpu.CompilerParams(
            use_tc_tiling_on_sc=False, needs_layout_passes=False),
    )
    def gather_k(y_hbm, idx_hbm, out_hbm, idx_a, idx_b, rows_a, rows_b,
                 tr_a, tr_b, sem_a, sem_b):
        wid = lax.axis_index("s") * _NC + lax.axis_index("c")
        base = wid * per_w
        iota = lax.iota(jnp.int32, 16)

        def fire(idx_v, rows_v, sem, g):
            b0 = base + g * B_CH
            pltpu.sync_copy(idx_hbm.at[pl.ds(b0, B_CH)], idx_v)
            for j in range(B_CH):
                pltpu.async_copy(
                    y_hbm.at[idx_v.at[j]], rows_v.at[pl.ds(j * L, L)], sem)

        def drain_store(rows_v, tr_v, sem, g):
            b0 = base + g * B_CH
            pltpu.make_async_copy(y_hbm.at[pl.ds(0, B_CH * L)], rows_v, sem).wait()

            def tbody(l, carry):
                row_idx = iota * L + l
                for d in range(DIM):
                    tr_v[l, d] = plsc.load_gather(
                        rows_v, [row_idx, jnp.full((16,), d, jnp.int32)])
                return carry

            lax.fori_loop(0, L, tbody, 0)
            pltpu.sync_copy(
                tr_v, out_hbm.at[pl.ds(0, L), pl.ds(0, DIM), pl.ds(b0, B_CH)])

        fire(idx_a, rows_a, sem_a, 0)

        def pair(p, carry):
            g0 = 2 * p
            fire(idx_b, rows_b, sem_b, g0 + 1)
            drain_store(rows_a, tr_a, sem_a, g0)

            @pl.when(g0 + 2 < n_ch)
            def _():
                fire(idx_a, rows_a, sem_a, g0 + 2)

            drain_store(rows_b, tr_b, sem_b, g0 + 1)
            return carry

        lax.fori_loop(0, n_ch // 2, pair, 0)

    return gather_k


def kernel(inputs, table, W, b):
    B, L = inputs.shape
    y128 = _transform_table_t(jnp.swapaxes(table, 0, 1), W, b)
    v4 = y128.reshape(4 * NUM_EMB, DIM)
    idx4 = inputs.astype(jnp.int32) * 4
    out_t = _make_gather3d(B, L)(v4, idx4)
    return jnp.transpose(out_t, (2, 0, 1))


# speedup vs baseline: 1.3020x; 1.3020x over previous
import functools

import jax
import jax.numpy as jnp
from jax import lax
from jax.experimental import pallas as pl
from jax.experimental.pallas import tpu as pltpu
from jax.experimental.pallas import tpu_sc as plsc

NUM_EMB = 1000000
DIM = 32
ROW_BLK = 8000

_info = plsc.get_sparse_core_info()
_NC, _NS = _info.num_cores, _info.num_subcores
_NW = _NC * _NS

B_CH = 16


def _transform_body(xt_ref, w_ref, b_ref, y_ref):
    h = jax.lax.dot_general(
        xt_ref[...], w_ref[...], (((0,), (0,)), ((), ())),
        preferred_element_type=jnp.float32)
    h = h + b_ref[...]
    norm = jnp.sqrt(jnp.sum(h * h, axis=-1, keepdims=True))
    y = h / norm
    y_ref[...] = jnp.concatenate(
        [y, jnp.zeros((y.shape[0], 128 - DIM), jnp.float32)], axis=1)


COL_BLK = 16384


def _transform_table_t(table_t, W, b):
    return pl.pallas_call(
        _transform_body,
        grid=(pl.cdiv(NUM_EMB, COL_BLK),),
        in_specs=[
            pl.BlockSpec((DIM, COL_BLK), lambda i: (0, i)),
            pl.BlockSpec((DIM, DIM), lambda i: (0, 0)),
            pl.BlockSpec((1, DIM), lambda i: (0, 0)),
        ],
        out_specs=pl.BlockSpec((COL_BLK, 128), lambda i: (i, 0)),
        out_shape=jax.ShapeDtypeStruct((NUM_EMB, 128), jnp.float32),
    )(table_t, W, b.reshape(1, DIM))


def _make_gather3d(B, L):
    per_w = B // _NW
    n_ch = per_w // B_CH
    mesh = plsc.VectorSubcoreMesh(core_axis_name="c", subcore_axis_name="s")

    @functools.partial(
        pl.kernel,
        mesh=mesh,
        out_type=jax.ShapeDtypeStruct((B, L, DIM), jnp.float32),
        scratch_types=[
            pltpu.VMEM((B_CH, L), jnp.int32),
            pltpu.VMEM((B_CH, L), jnp.int32),
            pltpu.VMEM((B_CH, L, DIM), jnp.float32),
            pltpu.VMEM((B_CH, L, DIM), jnp.float32),
            pltpu.SemaphoreType.DMA,
            pltpu.SemaphoreType.DMA,
        ],
        compiler_params=pltpu.CompilerParams(use_tc_tiling_on_sc=False),
    )
    def gather_k(y_hbm, idx_hbm, out_hbm, idx_a, idx_b, rows_a, rows_b,
                 sem_a, sem_b):
        wid = lax.axis_index("s") * _NC + lax.axis_index("c")
        base = wid * per_w

        def fire(idx_v, rows_v, sem, g):
            b0 = base + g * B_CH
            pltpu.sync_copy(idx_hbm.at[pl.ds(b0, B_CH)], idx_v)
            for j in range(B_CH):
                pltpu.async_copy(y_hbm.at[idx_v.at[j]], rows_v.at[j], sem)

        def drain_store(rows_v, sem, g):
            b0 = base + g * B_CH
            pltpu.make_async_copy(out_hbm.at[pl.ds(b0, B_CH)], rows_v, sem).wait()
            pltpu.sync_copy(rows_v, out_hbm.at[pl.ds(b0, B_CH)])

        fire(idx_a, rows_a, sem_a, 0)

        def pair(p, carry):
            g0 = 2 * p
            fire(idx_b, rows_b, sem_b, g0 + 1)
            drain_store(rows_a, sem_a, g0)

            @pl.when(g0 + 2 < n_ch)
            def _():
                fire(idx_a, rows_a, sem_a, g0 + 2)

            drain_store(rows_b, sem_b, g0 + 1)
            return carry

        lax.fori_loop(0, n_ch // 2, pair, 0)

    return gather_k


def kernel(inputs, table, W, b):
    B, L = inputs.shape
    y128 = _transform_table_t(jnp.swapaxes(table, 0, 1), W, b)
    v4 = y128.reshape(4 * NUM_EMB, DIM)
    idx4 = inputs.astype(jnp.int32) * 4
    out = _make_gather3d(B, L)(v4, idx4)
    return out
